# baseline (speedup 1.0000x reference)
import jax
import jax.numpy as jnp
from jax import lax
from jax.experimental import pallas as pl
from jax.experimental.pallas import tpu as pltpu
from jax.experimental.pallas import tpu_sc as plsc

_NC = 2
_NS = 16
_NW = _NC * _NS

_GRP = 16
_FIRE = 13
_NB_CHUNK = 16


def _repack_body(src_ref, out_ref):
    q = pl.program_id(1)
    d = src_ref.shape[1]
    for k in range(128 // d):
        @pl.when(q == k)
        def _():
            out_ref[:, k * d:(k + 1) * d] = src_ref[...]


def _repack_table(table):
    n_rows, d = table.shape
    pack = 128 // d
    q_rows = n_rows // pack
    bbt = 2000
    nblk = q_rows // bbt
    return pl.pallas_call(
        _repack_body,
        grid=(nblk, pack),
        in_specs=[pl.BlockSpec((bbt, d), lambda i, q: (i + q * nblk, 0))],
        out_specs=pl.BlockSpec((bbt, 128), lambda i, q: (i, 0)),
        out_shape=jax.ShapeDtypeStruct((q_rows, 128), jnp.float32),
    )(table)


def _make_sc_gather(B: int, n_cat: int, d: int):
    rows_per_b = n_cat
    chunk = _NB_CHUNK * rows_per_b
    b_per_w = B // _NW
    n_chunks = b_per_w // _NB_CHUNK
    n_grp = chunk // _GRP
    dc = n_cat * d

    def body(t128_hbm, gidx_hbm, qd_hbm, srow_hbm, scol_hbm, out_hbm,
             gv_v, qd_v, srow_v, scol_v, rows_v, sel_v, sem):
        wid = lax.axis_index("s") * _NC + lax.axis_index("c")
        b_base = wid * b_per_w
        pltpu.sync_copy(srow_hbm, srow_v)
        pltpu.sync_copy(scol_hbm, scol_v)
        iota = lax.iota(jnp.int32, _GRP)

        def chunk_body(c, _):
            b0 = b_base + c * _NB_CHUNK
            off = pl.multiple_of(b0 * rows_per_b, chunk)
            pltpu.sync_copy(gidx_hbm.at[pl.ds(off, chunk)], gv_v)
            pltpu.sync_copy(qd_hbm.at[pl.ds(off, chunk)], qd_v)
            for g0 in range(0, n_grp, _FIRE):
                cps = []
                for g in range(g0, min(g0 + _FIRE, n_grp)):
                    gv = gv_v[pl.ds(g * _GRP, _GRP)]
                    cps.append(
                        pltpu.async_copy(
                            t128_hbm.at[gv],
                            rows_v.at[pl.ds(g * _GRP, _GRP)],
                            sem,
                        )
                    )
                for cp in cps:
                    cp.wait()

            def sel_body(g, _):
                rid = iota + g * _GRP
                cid0 = qd_v[pl.ds(g * _GRP, _GRP)]
                sr = srow_v[pl.ds(g * _GRP, _GRP)]
                sc0 = scol_v[pl.ds(g * _GRP, _GRP)]
                for cc in range(d):
                    vals = plsc.load_gather(rows_v, [rid, cid0 + cc])
                    plsc.store_scatter(sel_v, [sr, sc0 + cc], vals)
                return 0

            lax.fori_loop(0, n_grp, sel_body, 0)
            pltpu.sync_copy(sel_v, out_hbm.at[pl.ds(b0, _NB_CHUNK)])
            return 0

        lax.fori_loop(0, n_chunks, chunk_body, 0)

    mesh = plsc.VectorSubcoreMesh(
        core_axis_name="c", subcore_axis_name="s", num_cores=_NC, num_subcores=_NS
    )
    return pl.kernel(
        body,
        out_type=jax.ShapeDtypeStruct((B, dc), jnp.float32),
        mesh=mesh,
        compiler_params=pltpu.CompilerParams(needs_layout_passes=False),
        scratch_types=[
            pltpu.VMEM((chunk,), jnp.int32),
            pltpu.VMEM((chunk,), jnp.int32),
            pltpu.VMEM((chunk,), jnp.int32),
            pltpu.VMEM((chunk,), jnp.int32),
            pltpu.VMEM((chunk, 128), jnp.float32),
            pltpu.VMEM((_NB_CHUNK, dc), jnp.float32),
            pltpu.SemaphoreType.DMA,
        ],
    )


def _tc_body(xnt_ref, wt_ref, nbt_ref, cat_ref, cbt_ref, out_ref):
    num2t = jnp.dot(
        wt_ref[...],
        xnt_ref[...],
        preferred_element_type=jnp.float32,
        precision=jax.lax.Precision.HIGHEST,
    ) + nbt_ref[...]
    cat2t = cat_ref[...].T + cbt_ref[...]
    out_ref[...] = jnp.concatenate([num2t, cat2t], axis=0)


def kernel(x_num, x_cat, num_w, num_b, cat_table, cat_bias):
    B, n_num, n_bins = x_num.shape
    n_cat = x_cat.shape[1]
    d_emb = cat_table.shape[1]
    card = cat_table.shape[0] // n_cat

    t128 = _repack_table(cat_table)
    q_rows = t128.shape[0]

    offsets = (jnp.arange(n_cat, dtype=jnp.int32) * card)[None]
    idx = (x_cat.astype(jnp.int32) + offsets).reshape(-1)
    q = idx // q_rows
    gidx = idx - q * q_rows
    qd = q * d_emb
    chunk = _NB_CHUNK * n_cat
    j = jnp.arange(chunk, dtype=jnp.int32)
    srow = j // n_cat
    scol = (j - srow * n_cat) * d_emb
    gather = _make_sc_gather(B, n_cat, d_emb)
    cat2 = gather(t128, gidx, qd, srow, scol)

    dn = n_num * n_bins
    dt = (n_num + n_cat) * d_emb
    eye = jnp.eye(n_num, dtype=jnp.float32)
    w_blk = (eye[:, None, :, None] * num_w[:, :, None, :]).reshape(
        dn, n_num * d_emb
    )
    xnt = x_num.reshape(B, dn).T
    bb = 512
    zt = pl.pallas_call(
        _tc_body,
        grid=(B // bb,),
        in_specs=[
            pl.BlockSpec((dn, bb), lambda i: (0, i)),
            pl.BlockSpec((n_num * d_emb, dn), lambda i: (0, 0)),
            pl.BlockSpec((n_num * d_emb, 1), lambda i: (0, 0)),
            pl.BlockSpec((bb, n_cat * d_emb), lambda i: (i, 0)),
            pl.BlockSpec((n_cat * d_emb, 1), lambda i: (0, 0)),
        ],
        out_specs=pl.BlockSpec((dt, bb), lambda i: (0, i)),
        out_shape=jax.ShapeDtypeStruct((dt, B), jnp.float32),
    )(
        xnt,
        w_blk.T,
        num_b.reshape(n_num * d_emb, 1),
        cat2,
        cat_bias.reshape(n_cat * d_emb, 1),
    )
    return zt.reshape(n_num + n_cat, d_emb, B).transpose(2, 0, 1)


# speedup vs baseline: 2.2038x; 2.2038x over previous
import jax
import jax.numpy as jnp
from jax import lax
from jax.experimental import pallas as pl
from jax.experimental.pallas import tpu as pltpu
from jax.experimental.pallas import tpu_sc as plsc

_NC = 2
_NS = 16
_NW = _NC * _NS

_GRP = 16
_FIRE = 13
_NB_CHUNK = 16


_BBT = 4096


def _repack_body(src_ref, out_ref):
    d = src_ref.shape[0]
    pack = 128 // d
    out_ref[...] = jnp.concatenate(
        [src_ref[:, k * _BBT:(k + 1) * _BBT].T for k in range(pack)], axis=1
    )


def _repack_table(table_t):
    d, n_rows = table_t.shape
    pack = 128 // d
    nblk = (n_rows + pack * _BBT - 1) // (pack * _BBT)
    return pl.pallas_call(
        _repack_body,
        grid=(nblk,),
        in_specs=[pl.BlockSpec((d, pack * _BBT), lambda i: (0, i))],
        out_specs=pl.BlockSpec((_BBT, 128), lambda i: (i, 0)),
        out_shape=jax.ShapeDtypeStruct((nblk * _BBT, 128), jnp.float32),
    )(table_t)


def _make_sc_gather(B: int, n_cat: int, d: int):
    rows_per_b = n_cat
    chunk = _NB_CHUNK * rows_per_b
    b_per_w = B // _NW
    n_chunks = b_per_w // _NB_CHUNK
    n_grp = chunk // _GRP
    dc = n_cat * d

    def body(t128_hbm, gidx_hbm, qd_hbm, srow_hbm, scol_hbm, out_hbm,
             gv_v, qd_v, srow_v, scol_v, rows_v, sel_v, sem):
        wid = lax.axis_index("s") * _NC + lax.axis_index("c")
        b_base = wid * b_per_w
        pltpu.sync_copy(srow_hbm, srow_v)
        pltpu.sync_copy(scol_hbm, scol_v)
        iota = lax.iota(jnp.int32, _GRP)

        def chunk_body(c, _):
            b0 = b_base + c * _NB_CHUNK
            off = pl.multiple_of(b0 * rows_per_b, chunk)
            pltpu.sync_copy(gidx_hbm.at[pl.ds(off, chunk)], gv_v)
            pltpu.sync_copy(qd_hbm.at[pl.ds(off, chunk)], qd_v)
            for g0 in range(0, n_grp, _FIRE):
                cps = []
                for g in range(g0, min(g0 + _FIRE, n_grp)):
                    gv = gv_v[pl.ds(g * _GRP, _GRP)]
                    cps.append(
                        pltpu.async_copy(
                            t128_hbm.at[gv],
                            rows_v.at[pl.ds(g * _GRP, _GRP)],
                            sem,
                        )
                    )
                for cp in cps:
                    cp.wait()

            @plsc.parallel_loop(0, n_grp, 1, unroll=2)
            def sel_body(g):
                rid = iota + g * _GRP
                cid0 = qd_v[pl.ds(g * _GRP, _GRP)]
                sr = srow_v[pl.ds(g * _GRP, _GRP)]
                sc0 = scol_v[pl.ds(g * _GRP, _GRP)]
                for cc in range(d):
                    vals = plsc.load_gather(rows_v, [rid, cid0 + cc])
                    plsc.store_scatter(sel_v, [sr, sc0 + cc], vals)
            pltpu.sync_copy(sel_v, out_hbm.at[pl.ds(b0, _NB_CHUNK)])
            return 0

        lax.fori_loop(0, n_chunks, chunk_body, 0)

    mesh = plsc.VectorSubcoreMesh(
        core_axis_name="c", subcore_axis_name="s", num_cores=_NC, num_subcores=_NS
    )
    return pl.kernel(
        body,
        out_type=jax.ShapeDtypeStruct((B, dc), jnp.float32),
        mesh=mesh,
        compiler_params=pltpu.CompilerParams(needs_layout_passes=False),
        scratch_types=[
            pltpu.VMEM((chunk,), jnp.int32),
            pltpu.VMEM((chunk,), jnp.int32),
            pltpu.VMEM((chunk,), jnp.int32),
            pltpu.VMEM((chunk,), jnp.int32),
            pltpu.VMEM((chunk, 128), jnp.float32),
            pltpu.VMEM((_NB_CHUNK, dc), jnp.float32),
            pltpu.SemaphoreType.DMA,
        ],
    )


def _tc_body(xnt_ref, wt_ref, nbt_ref, cat_ref, cbt_ref, out_ref):
    num2t = jnp.dot(
        wt_ref[...],
        xnt_ref[...],
        preferred_element_type=jnp.float32,
        precision=jax.lax.Precision.HIGHEST,
    ) + nbt_ref[...]
    cat2t = cat_ref[...].T + cbt_ref[...]
    out_ref[...] = jnp.concatenate([num2t, cat2t], axis=0)


def kernel(x_num, x_cat, num_w, num_b, cat_table, cat_bias):
    B, n_num, n_bins = x_num.shape
    n_cat = x_cat.shape[1]
    d_emb = cat_table.shape[1]
    card = cat_table.shape[0] // n_cat

    t128 = _repack_table(cat_table.T)
    q_rows = t128.shape[0]

    offsets = (jnp.arange(n_cat, dtype=jnp.int32) * card)[None]
    idx = (x_cat.astype(jnp.int32) + offsets).reshape(-1)
    pack = 128 // d_emb
    blk = idx // (pack * _BBT)
    rem = idx - blk * (pack * _BBT)
    k = rem // _BBT
    gidx = blk * _BBT + (rem - k * _BBT)
    qd = k * d_emb
    chunk = _NB_CHUNK * n_cat
    j = jnp.arange(chunk, dtype=jnp.int32)
    srow = j // n_cat
    scol = (j - srow * n_cat) * d_emb
    gather = _make_sc_gather(B, n_cat, d_emb)
    cat2 = gather(t128, gidx, qd, srow, scol)

    dn = n_num * n_bins
    dt = (n_num + n_cat) * d_emb
    eye = jnp.eye(n_num, dtype=jnp.float32)
    w_blk = (eye[:, None, :, None] * num_w[:, :, None, :]).reshape(
        dn, n_num * d_emb
    )
    xnt = x_num.reshape(B, dn).T
    bb = 512
    zt = pl.pallas_call(
        _tc_body,
        grid=(B // bb,),
        in_specs=[
            pl.BlockSpec((dn, bb), lambda i: (0, i)),
            pl.BlockSpec((n_num * d_emb, dn), lambda i: (0, 0)),
            pl.BlockSpec((n_num * d_emb, 1), lambda i: (0, 0)),
            pl.BlockSpec((bb, n_cat * d_emb), lambda i: (i, 0)),
            pl.BlockSpec((n_cat * d_emb, 1), lambda i: (0, 0)),
        ],
        out_specs=pl.BlockSpec((dt, bb), lambda i: (0, i)),
        out_shape=jax.ShapeDtypeStruct((dt, B), jnp.float32),
    )(
        xnt,
        w_blk.T,
        num_b.reshape(n_num * d_emb, 1),
        cat2,
        cat_bias.reshape(n_cat * d_emb, 1),
    )
    return zt.reshape(n_num + n_cat, d_emb, B).transpose(2, 0, 1)
